# baseline (speedup 1.0000x reference)
import functools

import jax
import jax.numpy as jnp
from jax import lax
from jax.experimental import pallas as pl
from jax.experimental.pallas import tpu as pltpu
from jax.experimental.pallas import tpu_sc as plsc

B, C, H, W, K = 16, 2, 128, 128, 128
HW = H * W
L = 16
NCHUNK = K // L

_mesh = plsc.VectorSubcoreMesh(core_axis_name="c", subcore_axis_name="s")


@functools.partial(
    pl.kernel,
    mesh=_mesh,
    out_type=jax.ShapeDtypeStruct((L,), jnp.float32),
    scratch_types=[
        pltpu.VMEM((2, K), jnp.int32),
        pltpu.VMEM((K,), jnp.float32),
        pltpu.VMEM((K,), jnp.int32),
        pltpu.VMEM((K,), jnp.int32),
        pltpu.VMEM((K,), jnp.float32),
        pltpu.VMEM((K,), jnp.float32),
        pltpu.VMEM((L,), jnp.float32),
        pltpu.VMEM((L,), jnp.float32),
        pltpu.VMEM_SHARED((L,), jnp.float32),
        pltpu.SemaphoreType.DMA,
    ],
)
def _confidence_loss_sc(flat_hbm, ind_hbm, mask_hbm, tgt_hbm, out_hbm,
                        im_v, tgt_v, idx0_v, idx1_v, p0_v, p1_v, stage_v,
                        acc_v, shared_acc, sem0):
    c = lax.axis_index("c")
    s = lax.axis_index("s")

    @pl.when((c == 0) & (s == 0))
    def _init():
        stage_v[...] = jnp.zeros((L,), jnp.float32)
        pltpu.sync_copy(stage_v, shared_acc)

    @pl.when(c == 0)
    def _work():
        base = s * K
        cp_ind = pltpu.async_copy(ind_hbm.at[pl.ds(base, K)], im_v.at[0],
                                  sem0)
        cp_msk = pltpu.async_copy(mask_hbm.at[pl.ds(base, K)], im_v.at[1],
                                  sem0)
        cp_tgt = pltpu.async_copy(tgt_hbm.at[pl.ds(base, K)], tgt_v, sem0)
        cp_ind.wait()
        cp_msk.wait()
        cp_tgt.wait()
        base0 = s * (C * HW)
        for j in range(NCHUNK):
            sl = pl.ds(j * L, L)
            iv = im_v[0, sl]
            idx0_v[sl] = iv + base0
            idx1_v[sl] = iv + (base0 + HW)
        cp0 = pltpu.async_copy(flat_hbm.at[idx0_v], p0_v, sem0)
        cp1 = pltpu.async_copy(flat_hbm.at[idx1_v], p1_v, sem0)
        cp0.wait()
        cp1.wait()
        acc = jnp.zeros((L,), jnp.float32)
        for j in range(NCHUNK):
            sl = pl.ds(j * L, L)
            p0 = p0_v[sl]
            p1 = p1_v[sl]
            m = im_v[1, sl].astype(jnp.float32)
            t = tgt_v[sl]
            a = jnp.abs(p0 - t)
            conf = 1.0 - jnp.exp(-a / t)
            acc = acc + m * (a + jnp.abs(p1 - conf))
        stage_v[...] = acc

    plsc.subcore_barrier()

    @pl.when(c == 0)
    def _accumulate():
        lane_ids = lax.iota(jnp.int32, L)
        pltpu.sync_copy(stage_v, shared_acc.at[lane_ids], add=True)

    plsc.subcore_barrier()

    @pl.when((c == 0) & (s == 0))
    def _reduce():
        pltpu.sync_copy(shared_acc, acc_v)
        tot = acc_v[...]
        total = jnp.float32(0.0)
        for i in range(L):
            total = total + tot[i]
        total = total * (1.0 / (B * K))
        acc_v[...] = jnp.full((L,), total, jnp.float32)
        pltpu.sync_copy(acc_v, out_hbm)


def kernel(output, mask, ind, target):
    flat = output.reshape(-1)
    ind_flat = ind.reshape(-1)
    mask_flat = mask.reshape(-1)
    tgt_flat = target.reshape(-1)
    out = _confidence_loss_sc(flat, ind_flat, mask_flat, tgt_flat)
    return out[0]


# speedup vs baseline: 1.0798x; 1.0798x over previous
import functools

import jax
import jax.numpy as jnp
from jax import lax
from jax.experimental import pallas as pl
from jax.experimental.pallas import tpu as pltpu
from jax.experimental.pallas import tpu_sc as plsc

B, C, H, W, K = 16, 2, 128, 128, 128
HW = H * W
L = 16
NCHUNK = K // L

_mesh = plsc.VectorSubcoreMesh(core_axis_name="c", subcore_axis_name="s",
                               num_cores=1)


@functools.partial(
    pl.kernel,
    mesh=_mesh,
    out_type=jax.ShapeDtypeStruct((L,), jnp.float32),
    scratch_types=[
        pltpu.VMEM((2, K), jnp.int32),
        pltpu.VMEM((K,), jnp.float32),
        pltpu.VMEM((K,), jnp.int32),
        pltpu.VMEM((K,), jnp.int32),
        pltpu.VMEM((K,), jnp.float32),
        pltpu.VMEM((K,), jnp.float32),
        pltpu.VMEM((L,), jnp.float32),
        pltpu.VMEM((L,), jnp.float32),
        pltpu.VMEM_SHARED((L,), jnp.float32),
        pltpu.SemaphoreType.DMA,
        pltpu.SemaphoreType.DMA,
        pltpu.SemaphoreType.DMA,
    ],
)
def _confidence_loss_sc(flat_hbm, ind_hbm, mask_hbm, tgt_hbm, out_hbm,
                        im_v, tgt_v, idx0_v, idx1_v, p0_v, p1_v, stage_v,
                        acc_v, shared_acc, sem0, sem1, sem2):
    c = lax.axis_index("c")
    s = lax.axis_index("s")

    @pl.when((c == 0) & (s == 0))
    def _init():
        stage_v[...] = jnp.zeros((L,), jnp.float32)
        pltpu.sync_copy(stage_v, shared_acc)

    @pl.when(c == 0)
    def _work():
        base = s * K
        cp_ind = pltpu.async_copy(ind_hbm.at[pl.ds(base, K)], im_v.at[0],
                                  sem0)
        cp_msk = pltpu.async_copy(mask_hbm.at[pl.ds(base, K)], im_v.at[1],
                                  sem1)
        cp_tgt = pltpu.async_copy(tgt_hbm.at[pl.ds(base, K)], tgt_v, sem2)
        cp_ind.wait()
        base0 = s * (C * HW)
        for j in range(NCHUNK):
            sl = pl.ds(j * L, L)
            iv = im_v[0, sl]
            idx0_v[sl] = iv + base0
            idx1_v[sl] = iv + (base0 + HW)
        cp0 = pltpu.async_copy(flat_hbm.at[idx0_v], p0_v, sem0)
        cp1 = pltpu.async_copy(flat_hbm.at[idx1_v], p1_v, sem1)
        cp_msk.wait()
        cp_tgt.wait()
        cp0.wait()
        cp1.wait()
        acc = jnp.zeros((L,), jnp.float32)
        for j in range(NCHUNK):
            sl = pl.ds(j * L, L)
            p0 = p0_v[sl]
            p1 = p1_v[sl]
            m = im_v[1, sl].astype(jnp.float32)
            t = tgt_v[sl]
            a = jnp.abs(p0 - t)
            conf = 1.0 - jnp.exp(-a / t)
            acc = acc + m * (a + jnp.abs(p1 - conf))
        stage_v[...] = acc

    plsc.subcore_barrier()

    @pl.when(c == 0)
    def _accumulate():
        lane_ids = lax.iota(jnp.int32, L)
        pltpu.sync_copy(stage_v, shared_acc.at[lane_ids], add=True)

    plsc.subcore_barrier()

    @pl.when((c == 0) & (s == 0))
    def _reduce():
        pltpu.sync_copy(shared_acc, acc_v)
        tot = acc_v[...]
        total = jnp.float32(0.0)
        for i in range(L):
            total = total + tot[i]
        total = total * (1.0 / (B * K))
        acc_v[...] = jnp.full((L,), total, jnp.float32)
        pltpu.sync_copy(acc_v, out_hbm)


def kernel(output, mask, ind, target):
    flat = output.reshape(-1)
    ind_flat = ind.reshape(-1)
    mask_flat = mask.reshape(-1)
    tgt_flat = target.reshape(-1)
    out = _confidence_loss_sc(flat, ind_flat, mask_flat, tgt_flat)
    return out[0]


# speedup vs baseline: 1.0887x; 1.0083x over previous
import functools

import jax
import jax.numpy as jnp
from jax import lax
from jax.experimental import pallas as pl
from jax.experimental.pallas import tpu as pltpu
from jax.experimental.pallas import tpu_sc as plsc

B, C, H, W, K = 16, 2, 128, 128, 128
HW = H * W
L = 16
NCHUNK = K // L

_mesh = plsc.VectorSubcoreMesh(core_axis_name="c", subcore_axis_name="s",
                               num_cores=1)


@functools.partial(
    pl.kernel,
    mesh=_mesh,
    out_type=jax.ShapeDtypeStruct((L,), jnp.float32),
    scratch_types=[
        pltpu.VMEM((2, K), jnp.int32),
        pltpu.VMEM((K,), jnp.float32),
        pltpu.VMEM((K,), jnp.int32),
        pltpu.VMEM((K,), jnp.int32),
        pltpu.VMEM((K,), jnp.float32),
        pltpu.VMEM((K,), jnp.float32),
        pltpu.VMEM((L,), jnp.float32),
        pltpu.VMEM((16 * L,), jnp.float32),
        pltpu.VMEM_SHARED((16 * L,), jnp.float32),
        pltpu.SemaphoreType.DMA,
        pltpu.SemaphoreType.DMA,
        pltpu.SemaphoreType.DMA,
    ],
)
def _confidence_loss_sc(flat_hbm, ind_hbm, mask_hbm, tgt_hbm, out_hbm,
                        im_v, tgt_v, idx0_v, idx1_v, p0_v, p1_v, stage_v,
                        all_v, shared_slots, sem0, sem1, sem2):
    c = lax.axis_index("c")
    s = lax.axis_index("s")

    @pl.when(c == 0)
    def _work():
        base = s * K
        cp_ind = pltpu.async_copy(ind_hbm.at[pl.ds(base, K)], im_v.at[0],
                                  sem0)
        cp_msk = pltpu.async_copy(mask_hbm.at[pl.ds(base, K)], im_v.at[1],
                                  sem1)
        cp_tgt = pltpu.async_copy(tgt_hbm.at[pl.ds(base, K)], tgt_v, sem2)
        cp_ind.wait()
        base0 = s * (C * HW)
        for j in range(NCHUNK):
            sl = pl.ds(j * L, L)
            iv = im_v[0, sl]
            idx0_v[sl] = iv + base0
            idx1_v[sl] = iv + (base0 + HW)
        cp0 = pltpu.async_copy(flat_hbm.at[idx0_v], p0_v, sem0)
        cp1 = pltpu.async_copy(flat_hbm.at[idx1_v], p1_v, sem1)
        cp_msk.wait()
        cp_tgt.wait()
        cp0.wait()
        cp1.wait()
        acc = jnp.zeros((L,), jnp.float32)
        for j in range(NCHUNK):
            sl = pl.ds(j * L, L)
            p0 = p0_v[sl]
            p1 = p1_v[sl]
            m = im_v[1, sl].astype(jnp.float32)
            t = tgt_v[sl]
            a = jnp.abs(p0 - t)
            conf = 1.0 - jnp.exp(-a / t)
            acc = acc + m * (a + jnp.abs(p1 - conf))
        stage_v[...] = acc
        pltpu.sync_copy(stage_v, shared_slots.at[pl.ds(s * L, L)])

    plsc.subcore_barrier()

    @pl.when((c == 0) & (s == 0))
    def _reduce():
        pltpu.sync_copy(shared_slots, all_v)
        tot = jnp.zeros((L,), jnp.float32)
        for i in range(16):
            tot = tot + all_v[pl.ds(i * L, L)]
        total = jnp.float32(0.0)
        for i in range(L):
            total = total + tot[i]
        total = total * (1.0 / (B * K))
        stage_v[...] = jnp.full((L,), total, jnp.float32)
        pltpu.sync_copy(stage_v, out_hbm)


def kernel(output, mask, ind, target):
    flat = output.reshape(-1)
    ind_flat = ind.reshape(-1)
    mask_flat = mask.reshape(-1)
    tgt_flat = target.reshape(-1)
    out = _confidence_loss_sc(flat, ind_flat, mask_flat, tgt_flat)
    return out[0]
